# baseline (speedup 1.0000x reference)
import functools

import jax

jax.config.update("jax_enable_x64", True)

import jax.numpy as jnp
from jax import lax
from jax.experimental import pallas as pl
from jax.experimental.pallas import tpu as pltpu
from jax.experimental.pallas import tpu_sc as plsc

_N = 65536
_K = int(_N * 0.2)
_C0 = _N - _K
_NT = 16
_CH = _N // _NT
_PL = _CH // 16
_HB = 20
_DUMP = 13112
_DSZ = _DUMP + 2048 + 8
_MIN32 = -2147483648


def _lanes():
    return lax.iota(jnp.int32, 16)


def _sortable(b):
    return jnp.where(b < 0, ~b, b ^ jnp.int32(_MIN32))


def _rank_body(x_hbm, sidx_hbm, hu_hbm, xf, ck, ci, hist, run, plist, wbuf,
               tloc, small, kbuf, ibuf, table, hubuf, sem):
    c = lax.axis_index("c")
    w = lax.axis_index("s")
    lanes = _lanes()
    base = w * _CH

    pltpu.sync_copy(x_hbm.at[c].at[pl.ds(base, _CH)], xf)

    def init_body(i, _):
        i = jnp.int32(i)
        xv = xf[pl.ds(i * 16, 16)]
        b = lax.bitcast_convert_type(xv, jnp.int32)
        ck[pl.ds(i * 16, 16)] = _sortable(b)
        ci[pl.ds(i * 16, 16)] = base + i * 16 + lanes
        return jnp.int32(0)

    lax.fori_loop(jnp.int32(0), jnp.int32(_CH // 16), init_body, jnp.int32(0))

    _SHIFTS = (0, 8, 16, 16)
    for p in range(4):
        sh = jnp.int32(_SHIFTS[p])

        def zero_body(i, _):
            i = jnp.int32(i)
            hist[pl.ds(i * 16, 16)] = jnp.zeros(16, jnp.int32)
            return jnp.int32(0)

        lax.fori_loop(jnp.int32(0), jnp.int32(_CH // 16), zero_body, jnp.int32(0))

        def h_body(i, _):
            i = jnp.int32(i)
            j = lanes * _PL + i
            key = plsc.load_gather(ck, [j])
            d = lax.shift_right_logical(key, sh) & 255
            bn = lanes * 256 + d
            cnt = plsc.load_gather(hist, [bn])
            plsc.store_scatter(hist, [bn], cnt + 1)
            return jnp.int32(0)

        lax.fori_loop(jnp.int32(0), jnp.int32(_PL), h_body, jnp.int32(0))

        def t_body(g, _):
            g = jnp.int32(g)
            acc = jnp.zeros(16, jnp.int32)
            for l in range(16):
                acc = acc + hist[pl.ds(l * 256 + g * 16, 16)]
            wbuf[pl.ds(g * 16, 16)] = acc
            return jnp.int32(0)

        lax.fori_loop(jnp.int32(0), jnp.int32(16), t_body, jnp.int32(0))
        pltpu.sync_copy(wbuf.at[pl.ds(0, 256)], table.at[pl.ds(w * 256, 256)])
        plsc.subcore_barrier()

        pltpu.sync_copy(table, tloc)

        carry = jnp.int32(0)
        for g in range(16):
            tot = jnp.zeros(16, jnp.int32)
            pri = jnp.zeros(16, jnp.int32)
            for t in range(16):
                v = tloc[pl.ds(t * 256 + g * 16, 16)]
                tot = tot + v
                pri = pri + jnp.where(jnp.int32(t) < w, v, jnp.int32(0))
            ex = plsc.cumsum(tot) - tot + carry
            carry = carry + jnp.sum(tot, dtype=jnp.int32)
            rowbase = ex + pri
            lacc = jnp.zeros(16, jnp.int32)
            for l in range(16):
                run[pl.ds(l * 256 + g * 16, 16)] = rowbase + lacc
                lacc = lacc + hist[pl.ds(l * 256 + g * 16, 16)]

        last = p == 3

        def p_body(i, vacc):
            i = jnp.int32(i)
            j = lanes * _PL + i
            key = plsc.load_gather(ck, [j])
            d = lax.shift_right_logical(key, sh) & 255
            bn = lanes * 256 + d
            pos = plsc.load_gather(run, [bn])
            plsc.store_scatter(run, [bn], pos + 1)
            plsc.store_scatter(plist, [j], pos)
            if p == 1:
                idx16 = plsc.load_gather(ci, [j]) & jnp.int32(0xFFFF)
                plsc.store_scatter(wbuf, [j], (key & jnp.int32(-65536)) | idx16)
            elif p == 2:
                w2 = (lax.shift_right_logical(key, jnp.int32(8))
                      & jnp.int32(0x00FF0000)) | (key & jnp.int32(0xFFFF))
                plsc.store_scatter(wbuf, [j], w2)
            elif p == 3:
                eid = key & jnp.int32(0xFFFF)
                plsc.store_scatter(wbuf, [j], eid)
                vacc = vacc + jnp.where(pos == jnp.int32(_C0), eid, jnp.int32(0))
            return vacc

        vacc = lax.fori_loop(jnp.int32(0), jnp.int32(_PL), p_body,
                             jnp.zeros(16, jnp.int32))

        if p == 0:
            pltpu.sync_copy(ck, kbuf.at[plist])
            pltpu.sync_copy(ci, ibuf.at[plist])
            plsc.subcore_barrier()
            pltpu.sync_copy(kbuf.at[pl.ds(base, _CH)], ck)
            pltpu.sync_copy(ibuf.at[pl.ds(base, _CH)], ci)
        else:
            pltpu.sync_copy(wbuf, kbuf.at[plist])
            plsc.subcore_barrier()
            pltpu.sync_copy(kbuf.at[pl.ds(base, _CH)], ck)

    pltpu.sync_copy(ck, sidx_hbm.at[c].at[pl.ds(base, _CH)])
    ci, ck = ck, ci

    vpart = jnp.sum(vacc, dtype=jnp.int32)
    small[...] = jnp.where(lanes == 0, vpart, jnp.int32(0))
    pltpu.sync_copy(small.at[pl.ds(0, 8)], table.at[pl.ds(w * 8, 8)])
    plsc.subcore_barrier()
    pltpu.sync_copy(table.at[pl.ds(0, 128)], tloc.at[pl.ds(0, 128)])
    ecut = jnp.sum(plsc.load_gather(tloc, [lanes * 8]), dtype=jnp.int32)
    own = (ecut >= base) & (ecut < base + jnp.int32(_CH))
    eloc = jnp.where(own, ecut - base, jnp.int32(0))
    xv = plsc.load_gather(xf, [jnp.zeros(16, jnp.int32) + eloc])
    kv = _sortable(lax.bitcast_convert_type(xv, jnp.int32))
    kcut = jnp.where(own, jnp.sum(jnp.where(lanes == 0, kv, jnp.int32(0)),
                                  dtype=jnp.int32), jnp.int32(0))
    small[...] = jnp.where(lanes == 0, kcut, jnp.int32(0))
    pltpu.sync_copy(small.at[pl.ds(0, 8)], table.at[pl.ds(256 + w * 8, 8)])
    plsc.subcore_barrier()
    pltpu.sync_copy(table.at[pl.ds(256, 128)], tloc.at[pl.ds(0, 128)])
    vstar = jnp.sum(plsc.load_gather(tloc, [lanes * 8]), dtype=jnp.int32)
    vs = vstar ^ jnp.int32(_MIN32)

    def c_body(i, acc):
        i = jnp.int32(i)
        bv = lax.bitcast_convert_type(xf[pl.ds(i * 16, 16)], jnp.int32)
        kv = _sortable(bv) ^ jnp.int32(_MIN32)
        lt, le = acc
        lt = lt + jnp.where(kv < vs, jnp.int32(1), jnp.int32(0))
        le = le + jnp.where(kv <= vs, jnp.int32(1), jnp.int32(0))
        return (lt, le)

    lt, le = lax.fori_loop(jnp.int32(0), jnp.int32(_CH // 16), c_body,
                           (jnp.zeros(16, jnp.int32), jnp.zeros(16, jnp.int32)))
    sp = jnp.sum(lt, dtype=jnp.int32)
    ep = jnp.sum(le, dtype=jnp.int32)
    small[...] = jnp.where(lanes == 0, sp, jnp.where(lanes == 1, ep, jnp.int32(0)))
    pltpu.sync_copy(small.at[pl.ds(0, 8)], table.at[pl.ds(128 + w * 8, 8)])
    plsc.subcore_barrier()
    pltpu.sync_copy(table.at[pl.ds(128, 128)], tloc.at[pl.ds(0, 128)])
    s_all = jnp.sum(plsc.load_gather(tloc, [lanes * 8]), dtype=jnp.int32)
    e_all = jnp.sum(plsc.load_gather(tloc, [lanes * 8 + 1]), dtype=jnp.int32)
    blk = e_all - jnp.int32(_C0)

    def hu_body(i, _):
        i = jnp.int32(i)
        gpos = base + i * 16 + lanes
        in_hi = gpos >= e_all
        in_tie = (gpos >= s_all) & (gpos < s_all + blk)
        hm = jnp.where(in_hi | in_tie, jnp.int32(1), jnp.int32(0))
        u = (jnp.maximum(jnp.int32(0),
                         jnp.minimum(gpos, s_all + blk) - s_all)
             + jnp.maximum(jnp.int32(0), gpos - e_all))
        wv = u + hm * jnp.int32(1 << _HB)
        wbuf[pl.ds(i * 16, 16)] = wv
        return jnp.int32(0)

    lax.fori_loop(jnp.int32(0), jnp.int32(_CH // 16), hu_body, jnp.int32(0))
    pltpu.sync_copy(wbuf, hubuf.at[ci])
    plsc.subcore_barrier()

    @pl.when(c == 0)
    def _():
        pltpu.sync_copy(hubuf.at[pl.ds(base, _CH)], hu_hbm.at[pl.ds(base, _CH)])


def _sums_body(sidx_hbm, hu_hbm, s1p_hbm, sabp_hbm, av, bv, hv, l1, l2, l3,
               small, dtab, table, hu_s, tloc16, sem):
    c = lax.axis_index("c")
    w = lax.axis_index("s")
    lanes = _lanes()
    base = w * _CH
    cc = jnp.float32((_N - 1) / 2.0)
    mu = jnp.float32((_K - 1) / 2.0)

    @pl.when(c == 0)
    def _():
        pltpu.sync_copy(sidx_hbm.at[jnp.int32(0)].at[pl.ds(base, _CH)], av)
        pltpu.sync_copy(sidx_hbm.at[jnp.int32(1)].at[pl.ds(base, _CH)], bv)

        def s1_body(i, acc):
            i = jnp.int32(i)
            am = av[pl.ds(i * 16, 16)].astype(jnp.float32) - cc
            bm = bv[pl.ds(i * 16, 16)].astype(jnp.float32) - cc
            return acc + am * bm

        acc = lax.fori_loop(jnp.int32(0), jnp.int32(_CH // 16), s1_body, jnp.zeros(16, jnp.float32))
        part = jnp.sum(acc)
        fsm = jnp.where(lanes == 0, part, jnp.float32(0.0))
        small[...] = lax.bitcast_convert_type(fsm, jnp.int32)
        pltpu.sync_copy(small.at[pl.ds(0, 8)], s1p_hbm.at[pl.ds(w * 8, 8)])

    @pl.when(c == 1)
    def _():
        pltpu.sync_copy(sidx_hbm.at[jnp.int32(1)].at[pl.ds(base, _CH)], av)

        @pl.when(w == jnp.int32(0))
        def _():
            pltpu.sync_copy(hu_hbm, hu_s)

        plsc.subcore_barrier()
        pltpu.async_copy(hu_s.at[av], hv, sem).wait()

        def a_body(i, acc):
            i = jnp.int32(i)
            j = lanes * _PL + i
            hu = plsc.load_gather(hv, [j])
            return acc + lax.shift_right_logical(hu, jnp.int32(_HB))

        tot = lax.fori_loop(jnp.int32(0), jnp.int32(_PL), a_body, jnp.zeros(16, jnp.int32))
        tsum = jnp.sum(tot, dtype=jnp.int32)
        small[...] = jnp.where(lanes == 0, tsum, jnp.int32(0))
        pltpu.sync_copy(small.at[pl.ds(0, 8)], table.at[pl.ds(w * 8, 8)])
        plsc.subcore_barrier()
        pltpu.sync_copy(table.at[pl.ds(0, 128)], tloc16)
        tts = plsc.load_gather(tloc16, [lanes * 8])
        tile_off = jnp.sum(jnp.where(lanes < w, tts, jnp.int32(0)), dtype=jnp.int32)
        start = tile_off + plsc.cumsum(tot) - tot

        def b_body(i, run):
            i = jnp.int32(i)
            j = lanes * _PL + i
            hu = plsc.load_gather(hv, [j])
            ht = lax.shift_right_logical(hu, jnp.int32(_HB))
            u = hu & jnp.int32((1 << _HB) - 1)
            du = jnp.int32(_K - 1) - u
            spread = jnp.int32(_DUMP) + (j & 2047)
            hd = ht == 1
            plsc.store_scatter(l1, [j], jnp.where(hd, run, spread))
            plsc.store_scatter(l2, [j], du)
            plsc.store_scatter(l3, [j], jnp.where(hd, u, spread))
            return run + ht

        lax.fori_loop(jnp.int32(0), jnp.int32(_PL), b_body, start)
        pltpu.sync_copy(l2, dtab.at[l1])
        plsc.subcore_barrier()
        pltpu.sync_copy(dtab.at[l3], bv)

        def c_body(i, acc):
            i = jnp.int32(i)
            hu = hv[pl.ds(i * 16, 16)]
            g = bv[pl.ds(i * 16, 16)]
            ht = lax.shift_right_logical(hu, jnp.int32(_HB))
            u = hu & jnp.int32((1 << _HB) - 1)
            du = (jnp.int32(_K - 1) - u).astype(jnp.float32) - mu
            gm = g.astype(jnp.float32) - mu
            return acc + jnp.where(ht == 1, du * gm, jnp.float32(0.0))

        acc = lax.fori_loop(jnp.int32(0), jnp.int32(_CH // 16), c_body, jnp.zeros(16, jnp.float32))
        part = jnp.sum(acc)
        fsm = jnp.where(lanes == 0, part, jnp.float32(0.0))
        small[...] = lax.bitcast_convert_type(fsm, jnp.int32)
        pltpu.sync_copy(small.at[pl.ds(0, 8)], sabp_hbm.at[pl.ds(w * 8, 8)])


def _combine_kernel(s1_ref, sab_ref, o_ref):
    s1c = jnp.sum(s1_ref[...])
    sabc = jnp.sum(sab_ref[...])
    n = jnp.float32(_N)
    k = jnp.float32(_K)
    s2c = n * (n * n - 1.0) / 12.0
    avg_ic = 1.0 - 6.0 * (2.0 * s2c - 2.0 * s1c) / (n * (n * n - 1.0))
    var = k * (k * k - 1.0) / 12.0
    head_ic = sabc / var
    r0 = avg_ic
    r2 = 2.0 * head_ic
    rmin = jnp.minimum(jnp.minimum(r0, r2), 0.0)
    rmean = (r0 + r2) / 6.0
    o_ref[...] = jnp.full((8, 128), -rmin * 0.5 - rmean * 0.5, jnp.float32)


@jax.jit
def kernel(model_op, tgt_op):
    m = jnp.squeeze(model_op).astype(jnp.float32)
    t = jnp.squeeze(tgt_op).astype(jnp.float32)
    x2 = jnp.stack([m, t])

    mesh = plsc.VectorSubcoreMesh(core_axis_name="c", subcore_axis_name="s")
    rank_k = functools.partial(
        pl.kernel,
        out_type=[
            jax.ShapeDtypeStruct((2, _N), jnp.int32),
            jax.ShapeDtypeStruct((_N,), jnp.int32),
        ],
        mesh=mesh,
        compiler_params=pltpu.CompilerParams(needs_layout_passes=False),
        scratch_types=[
            pltpu.VMEM((_CH,), jnp.float32),
            pltpu.VMEM((_CH,), jnp.int32),
            pltpu.VMEM((_CH,), jnp.int32),
            pltpu.VMEM((4096,), jnp.int32),
            pltpu.VMEM((4096,), jnp.int32),
            pltpu.VMEM((_CH,), jnp.int32),
            pltpu.VMEM((_CH,), jnp.int32),
            pltpu.VMEM((4096,), jnp.int32),
            pltpu.VMEM((16,), jnp.int32),
            pltpu.VMEM_SHARED((_N,), jnp.int32),
            pltpu.VMEM_SHARED((_N,), jnp.int32),
            pltpu.VMEM_SHARED((4096,), jnp.int32),
            pltpu.VMEM_SHARED((_N,), jnp.int32),
            pltpu.SemaphoreType.DMA,
        ],
    )(_rank_body)
    sidx, hu = rank_k(x2)

    sums_k = functools.partial(
        pl.kernel,
        out_type=[
            jax.ShapeDtypeStruct((128,), jnp.int32),
            jax.ShapeDtypeStruct((128,), jnp.int32),
        ],
        mesh=mesh,
        compiler_params=pltpu.CompilerParams(needs_layout_passes=False),
        scratch_types=[
            pltpu.VMEM((_CH,), jnp.int32),
            pltpu.VMEM((_CH,), jnp.int32),
            pltpu.VMEM((_CH,), jnp.int32),
            pltpu.VMEM((_CH,), jnp.int32),
            pltpu.VMEM((_CH,), jnp.int32),
            pltpu.VMEM((_CH,), jnp.int32),
            pltpu.VMEM((16,), jnp.int32),
            pltpu.VMEM_SHARED((_DSZ,), jnp.int32),
            pltpu.VMEM_SHARED((4096,), jnp.int32),
            pltpu.VMEM_SHARED((_N,), jnp.int32),
            pltpu.VMEM((128,), jnp.int32),
            pltpu.SemaphoreType.DMA,
        ],
    )(_sums_body)
    s1p_i, sabp_i = sums_k(sidx, hu)
    s1p = lax.bitcast_convert_type(s1p_i, jnp.float32).reshape(1, 128)
    sabp = lax.bitcast_convert_type(sabp_i, jnp.float32).reshape(1, 128)

    out = pl.pallas_call(
        _combine_kernel,
        out_shape=jax.ShapeDtypeStruct((8, 128), jnp.float32),
    )(s1p, sabp)
    return out[0, 0]


# speedup vs baseline: 1.1207x; 1.1207x over previous
import functools

import jax

jax.config.update("jax_enable_x64", True)

import jax.numpy as jnp
from jax import lax
from jax.experimental import pallas as pl
from jax.experimental.pallas import tpu as pltpu
from jax.experimental.pallas import tpu_sc as plsc

_N = 65536
_K = int(_N * 0.2)
_C0 = _N - _K
_NT = 16
_CH = _N // _NT
_PL = _CH // 16
_HB = 20
_DUMP = 13112
_DSZ = _DUMP + 2048 + 8
_MIN32 = -2147483648


def _lanes():
    return lax.iota(jnp.int32, 16)


def _sortable(b):
    return jnp.where(b < 0, ~b, b ^ jnp.int32(_MIN32))


def _rank_body(x_hbm, sidx_hbm, hu_hbm, xf, ck, ci, hist, run, plist, wbuf,
               tloc, small, kbuf, ibuf, table, hubuf, sem):
    c = lax.axis_index("c")
    w = lax.axis_index("s")
    lanes = _lanes()
    base = w * _CH

    pltpu.sync_copy(x_hbm.at[c].at[pl.ds(base, _CH)], xf)

    def init_body(i, _):
        i = jnp.int32(i)
        xv = xf[pl.ds(i * 16, 16)]
        b = lax.bitcast_convert_type(xv, jnp.int32)
        ck[pl.ds(i * 16, 16)] = _sortable(b)
        ci[pl.ds(i * 16, 16)] = base + i * 16 + lanes
        return jnp.int32(0)

    lax.fori_loop(jnp.int32(0), jnp.int32(_CH // 16), init_body, jnp.int32(0))

    for p in range(4):
        sh = jnp.int32(8 * p)

        def zero_body(i, _):
            i = jnp.int32(i)
            for q in range(8):
                hist[pl.ds((i * 8 + q) * 16, 16)] = jnp.zeros(16, jnp.int32)
            return jnp.int32(0)

        lax.fori_loop(jnp.int32(0), jnp.int32(_CH // 128), zero_body,
                      jnp.int32(0))

        def h_body(i, _):
            i = jnp.int32(i)
            for q in range(4):
                j = lanes * _PL + (i * 4 + q)
                key = plsc.load_gather(ck, [j])
                d = lax.shift_right_logical(key, sh) & 255
                bn = lanes * 256 + d
                cnt = plsc.load_gather(hist, [bn])
                plsc.store_scatter(hist, [bn], cnt + 1)
            return jnp.int32(0)

        lax.fori_loop(jnp.int32(0), jnp.int32(_PL // 4), h_body, jnp.int32(0))

        def t_body(g, _):
            g = jnp.int32(g)
            acc = jnp.zeros(16, jnp.int32)
            for l in range(16):
                acc = acc + hist[pl.ds(l * 256 + g * 16, 16)]
            wbuf[pl.ds(g * 16, 16)] = acc
            return jnp.int32(0)

        lax.fori_loop(jnp.int32(0), jnp.int32(16), t_body, jnp.int32(0))
        pltpu.sync_copy(wbuf.at[pl.ds(0, 256)], table.at[pl.ds(w * 256, 256)])
        plsc.subcore_barrier()

        pltpu.sync_copy(table, tloc)

        carry = jnp.int32(0)
        for g in range(16):
            tot = jnp.zeros(16, jnp.int32)
            pri = jnp.zeros(16, jnp.int32)
            for t in range(16):
                v = tloc[pl.ds(t * 256 + g * 16, 16)]
                tot = tot + v
                pri = pri + jnp.where(jnp.int32(t) < w, v, jnp.int32(0))
            ex = plsc.cumsum(tot) - tot + carry
            carry = carry + jnp.sum(tot, dtype=jnp.int32)
            rowbase = ex + pri
            lacc = jnp.zeros(16, jnp.int32)
            for l in range(16):
                run[pl.ds(l * 256 + g * 16, 16)] = rowbase + lacc
                lacc = lacc + hist[pl.ds(l * 256 + g * 16, 16)]

        last = p == 3

        def p_body(i, vacc):
            i = jnp.int32(i)
            for q in range(4):
                j = lanes * _PL + (i * 4 + q)
                key = plsc.load_gather(ck, [j])
                d = lax.shift_right_logical(key, sh) & 255
                bn = lanes * 256 + d
                pos = plsc.load_gather(run, [bn])
                plsc.store_scatter(run, [bn], pos + 1)
                plsc.store_scatter(plist, [j], pos)
                if last:
                    vacc = vacc + jnp.where(pos == jnp.int32(_C0), key,
                                            jnp.int32(0))
            return vacc

        vacc = lax.fori_loop(jnp.int32(0), jnp.int32(_PL // 4), p_body,
                             jnp.zeros(16, jnp.int32))

        if not last:
            pltpu.sync_copy(ck, kbuf.at[plist])
        pltpu.sync_copy(ci, ibuf.at[plist])
        plsc.subcore_barrier()
        if not last:
            pltpu.sync_copy(kbuf.at[pl.ds(base, _CH)], ck)
        pltpu.sync_copy(ibuf.at[pl.ds(base, _CH)], ci)

    pltpu.sync_copy(ci, sidx_hbm.at[c].at[pl.ds(base, _CH)])

    vpart = jnp.sum(vacc, dtype=jnp.int32)
    small[...] = jnp.where(lanes == 0, vpart, jnp.int32(0))
    pltpu.sync_copy(small.at[pl.ds(0, 8)], table.at[pl.ds(w * 8, 8)])
    plsc.subcore_barrier()
    pltpu.sync_copy(table.at[pl.ds(0, 128)], tloc.at[pl.ds(0, 128)])
    vstar = jnp.sum(plsc.load_gather(tloc, [lanes * 8]), dtype=jnp.int32)
    vs = vstar ^ jnp.int32(_MIN32)

    def c_body(i, acc):
        i = jnp.int32(i)
        kv = ck[pl.ds(i * 16, 16)] ^ jnp.int32(_MIN32)
        lt, le = acc
        lt = lt + jnp.where(kv < vs, jnp.int32(1), jnp.int32(0))
        le = le + jnp.where(kv <= vs, jnp.int32(1), jnp.int32(0))
        return (lt, le)

    lt, le = lax.fori_loop(jnp.int32(0), jnp.int32(_CH // 16), c_body,
                           (jnp.zeros(16, jnp.int32), jnp.zeros(16, jnp.int32)))
    sp = jnp.sum(lt, dtype=jnp.int32)
    ep = jnp.sum(le, dtype=jnp.int32)
    small[...] = jnp.where(lanes == 0, sp, jnp.where(lanes == 1, ep, jnp.int32(0)))
    pltpu.sync_copy(small.at[pl.ds(0, 8)], table.at[pl.ds(128 + w * 8, 8)])
    plsc.subcore_barrier()
    pltpu.sync_copy(table.at[pl.ds(128, 128)], tloc.at[pl.ds(0, 128)])
    s_all = jnp.sum(plsc.load_gather(tloc, [lanes * 8]), dtype=jnp.int32)
    e_all = jnp.sum(plsc.load_gather(tloc, [lanes * 8 + 1]), dtype=jnp.int32)
    blk = e_all - jnp.int32(_C0)

    def hu_body(i, _):
        i = jnp.int32(i)
        gpos = base + i * 16 + lanes
        in_hi = gpos >= e_all
        in_tie = (gpos >= s_all) & (gpos < s_all + blk)
        hm = jnp.where(in_hi | in_tie, jnp.int32(1), jnp.int32(0))
        u = (jnp.maximum(jnp.int32(0),
                         jnp.minimum(gpos, s_all + blk) - s_all)
             + jnp.maximum(jnp.int32(0), gpos - e_all))
        wv = u + hm * jnp.int32(1 << _HB)
        wbuf[pl.ds(i * 16, 16)] = wv
        return jnp.int32(0)

    lax.fori_loop(jnp.int32(0), jnp.int32(_CH // 16), hu_body, jnp.int32(0))
    pltpu.sync_copy(wbuf, hubuf.at[ci])
    plsc.subcore_barrier()

    @pl.when(c == 0)
    def _():
        pltpu.sync_copy(hubuf.at[pl.ds(base, _CH)], hu_hbm.at[pl.ds(base, _CH)])


def _sums_body(sidx_hbm, hu_hbm, s1p_hbm, sabp_hbm, av, bv, hv, l1, l2, l3,
               small, dtab, table, hu_s, tloc16, sem):
    c = lax.axis_index("c")
    w = lax.axis_index("s")
    lanes = _lanes()
    base = w * _CH
    cc = jnp.float32((_N - 1) / 2.0)
    mu = jnp.float32((_K - 1) / 2.0)

    @pl.when(c == 0)
    def _():
        pltpu.sync_copy(sidx_hbm.at[jnp.int32(0)].at[pl.ds(base, _CH)], av)
        pltpu.sync_copy(sidx_hbm.at[jnp.int32(1)].at[pl.ds(base, _CH)], bv)

        def s1_body(i, acc):
            i = jnp.int32(i)
            am = av[pl.ds(i * 16, 16)].astype(jnp.float32) - cc
            bm = bv[pl.ds(i * 16, 16)].astype(jnp.float32) - cc
            return acc + am * bm

        acc = lax.fori_loop(jnp.int32(0), jnp.int32(_CH // 16), s1_body, jnp.zeros(16, jnp.float32))
        part = jnp.sum(acc)
        fsm = jnp.where(lanes == 0, part, jnp.float32(0.0))
        small[...] = lax.bitcast_convert_type(fsm, jnp.int32)
        pltpu.sync_copy(small.at[pl.ds(0, 8)], s1p_hbm.at[pl.ds(w * 8, 8)])

    @pl.when(c == 1)
    def _():
        pltpu.sync_copy(sidx_hbm.at[jnp.int32(1)].at[pl.ds(base, _CH)], av)

        @pl.when(w == jnp.int32(0))
        def _():
            pltpu.sync_copy(hu_hbm, hu_s)

        plsc.subcore_barrier()
        pltpu.async_copy(hu_s.at[av], hv, sem).wait()

        def a_body(i, acc):
            i = jnp.int32(i)
            j = lanes * _PL + i
            hu = plsc.load_gather(hv, [j])
            return acc + lax.shift_right_logical(hu, jnp.int32(_HB))

        tot = lax.fori_loop(jnp.int32(0), jnp.int32(_PL), a_body, jnp.zeros(16, jnp.int32))
        tsum = jnp.sum(tot, dtype=jnp.int32)
        small[...] = jnp.where(lanes == 0, tsum, jnp.int32(0))
        pltpu.sync_copy(small.at[pl.ds(0, 8)], table.at[pl.ds(w * 8, 8)])
        plsc.subcore_barrier()
        pltpu.sync_copy(table.at[pl.ds(0, 128)], tloc16)
        tts = plsc.load_gather(tloc16, [lanes * 8])
        tile_off = jnp.sum(jnp.where(lanes < w, tts, jnp.int32(0)), dtype=jnp.int32)
        start = tile_off + plsc.cumsum(tot) - tot

        def b_body(i, run):
            i = jnp.int32(i)
            j = lanes * _PL + i
            hu = plsc.load_gather(hv, [j])
            ht = lax.shift_right_logical(hu, jnp.int32(_HB))
            u = hu & jnp.int32((1 << _HB) - 1)
            du = jnp.int32(_K - 1) - u
            spread = jnp.int32(_DUMP) + (j & 2047)
            hd = ht == 1
            plsc.store_scatter(l1, [j], jnp.where(hd, run, spread))
            plsc.store_scatter(l2, [j], du)
            plsc.store_scatter(l3, [j], jnp.where(hd, u, spread))
            return run + ht

        lax.fori_loop(jnp.int32(0), jnp.int32(_PL), b_body, start)
        pltpu.sync_copy(l2, dtab.at[l1])
        plsc.subcore_barrier()
        pltpu.sync_copy(dtab.at[l3], bv)

        def c_body(i, acc):
            i = jnp.int32(i)
            hu = hv[pl.ds(i * 16, 16)]
            g = bv[pl.ds(i * 16, 16)]
            ht = lax.shift_right_logical(hu, jnp.int32(_HB))
            u = hu & jnp.int32((1 << _HB) - 1)
            du = (jnp.int32(_K - 1) - u).astype(jnp.float32) - mu
            gm = g.astype(jnp.float32) - mu
            return acc + jnp.where(ht == 1, du * gm, jnp.float32(0.0))

        acc = lax.fori_loop(jnp.int32(0), jnp.int32(_CH // 16), c_body, jnp.zeros(16, jnp.float32))
        part = jnp.sum(acc)
        fsm = jnp.where(lanes == 0, part, jnp.float32(0.0))
        small[...] = lax.bitcast_convert_type(fsm, jnp.int32)
        pltpu.sync_copy(small.at[pl.ds(0, 8)], sabp_hbm.at[pl.ds(w * 8, 8)])


def _combine_kernel(s1_ref, sab_ref, o_ref):
    s1c = jnp.sum(s1_ref[...])
    sabc = jnp.sum(sab_ref[...])
    n = jnp.float32(_N)
    k = jnp.float32(_K)
    s2c = n * (n * n - 1.0) / 12.0
    avg_ic = 1.0 - 6.0 * (2.0 * s2c - 2.0 * s1c) / (n * (n * n - 1.0))
    var = k * (k * k - 1.0) / 12.0
    head_ic = sabc / var
    r0 = avg_ic
    r2 = 2.0 * head_ic
    rmin = jnp.minimum(jnp.minimum(r0, r2), 0.0)
    rmean = (r0 + r2) / 6.0
    o_ref[...] = jnp.full((8, 128), -rmin * 0.5 - rmean * 0.5, jnp.float32)


@jax.jit
def kernel(model_op, tgt_op):
    m = jnp.squeeze(model_op).astype(jnp.float32)
    t = jnp.squeeze(tgt_op).astype(jnp.float32)
    x2 = jnp.stack([m, t])

    mesh = plsc.VectorSubcoreMesh(core_axis_name="c", subcore_axis_name="s")
    rank_k = functools.partial(
        pl.kernel,
        out_type=[
            jax.ShapeDtypeStruct((2, _N), jnp.int32),
            jax.ShapeDtypeStruct((_N,), jnp.int32),
        ],
        mesh=mesh,
        compiler_params=pltpu.CompilerParams(needs_layout_passes=False),
        scratch_types=[
            pltpu.VMEM((_CH,), jnp.float32),
            pltpu.VMEM((_CH,), jnp.int32),
            pltpu.VMEM((_CH,), jnp.int32),
            pltpu.VMEM((4096,), jnp.int32),
            pltpu.VMEM((4096,), jnp.int32),
            pltpu.VMEM((_CH,), jnp.int32),
            pltpu.VMEM((_CH,), jnp.int32),
            pltpu.VMEM((4096,), jnp.int32),
            pltpu.VMEM((16,), jnp.int32),
            pltpu.VMEM_SHARED((_N,), jnp.int32),
            pltpu.VMEM_SHARED((_N,), jnp.int32),
            pltpu.VMEM_SHARED((4096,), jnp.int32),
            pltpu.VMEM_SHARED((_N,), jnp.int32),
            pltpu.SemaphoreType.DMA,
        ],
    )(_rank_body)
    sidx, hu = rank_k(x2)

    sums_k = functools.partial(
        pl.kernel,
        out_type=[
            jax.ShapeDtypeStruct((128,), jnp.int32),
            jax.ShapeDtypeStruct((128,), jnp.int32),
        ],
        mesh=mesh,
        compiler_params=pltpu.CompilerParams(needs_layout_passes=False),
        scratch_types=[
            pltpu.VMEM((_CH,), jnp.int32),
            pltpu.VMEM((_CH,), jnp.int32),
            pltpu.VMEM((_CH,), jnp.int32),
            pltpu.VMEM((_CH,), jnp.int32),
            pltpu.VMEM((_CH,), jnp.int32),
            pltpu.VMEM((_CH,), jnp.int32),
            pltpu.VMEM((16,), jnp.int32),
            pltpu.VMEM_SHARED((_DSZ,), jnp.int32),
            pltpu.VMEM_SHARED((4096,), jnp.int32),
            pltpu.VMEM_SHARED((_N,), jnp.int32),
            pltpu.VMEM((128,), jnp.int32),
            pltpu.SemaphoreType.DMA,
        ],
    )(_sums_body)
    s1p_i, sabp_i = sums_k(sidx, hu)
    s1p = lax.bitcast_convert_type(s1p_i, jnp.float32).reshape(1, 128)
    sabp = lax.bitcast_convert_type(sabp_i, jnp.float32).reshape(1, 128)

    out = pl.pallas_call(
        _combine_kernel,
        out_shape=jax.ShapeDtypeStruct((8, 128), jnp.float32),
    )(s1p, sabp)
    return out[0, 0]


# speedup vs baseline: 1.1353x; 1.0131x over previous
import functools

import jax

jax.config.update("jax_enable_x64", True)

import jax.numpy as jnp
from jax import lax
from jax.experimental import pallas as pl
from jax.experimental.pallas import tpu as pltpu
from jax.experimental.pallas import tpu_sc as plsc

_N = 65536
_K = int(_N * 0.2)
_C0 = _N - _K
_NT = 16
_CH = _N // _NT
_PL = _CH // 16
_HB = 20
_DUMP = 13112
_DSZ = _DUMP + 2048 + 8
_MIN32 = -2147483648


def _lanes():
    return lax.iota(jnp.int32, 16)


def _sortable(b):
    return jnp.where(b < 0, ~b, b ^ jnp.int32(_MIN32))


def _rank_body(x_hbm, sidx_hbm, hu_hbm, xf, ck, ci, hist, run, plist, wbuf,
               tloc, small, kbuf, ibuf, table, hubuf, sem):
    c = lax.axis_index("c")
    w = lax.axis_index("s")
    lanes = _lanes()
    base = w * _CH

    pltpu.sync_copy(x_hbm.at[c].at[pl.ds(base, _CH)], xf)

    def init_body(i, _):
        i = jnp.int32(i)
        for q in range(4):
            i4 = i * 4 + q
            xv = xf[pl.ds(i4 * 16, 16)]
            b = lax.bitcast_convert_type(xv, jnp.int32)
            ck[pl.ds(i4 * 16, 16)] = _sortable(b)
            ci[pl.ds(i4 * 16, 16)] = base + i4 * 16 + lanes
        return jnp.int32(0)

    lax.fori_loop(jnp.int32(0), jnp.int32(_CH // 64), init_body, jnp.int32(0))

    for p in range(4):
        sh = jnp.int32(8 * p)

        def zero_body(i, _):
            i = jnp.int32(i)
            for q in range(8):
                hist[pl.ds((i * 8 + q) * 16, 16)] = jnp.zeros(16, jnp.int32)
            return jnp.int32(0)

        lax.fori_loop(jnp.int32(0), jnp.int32(_CH // 128), zero_body,
                      jnp.int32(0))

        def h_body(i, _):
            i = jnp.int32(i)
            for q in range(8):
                j = lanes * _PL + (i * 8 + q)
                key = plsc.load_gather(ck, [j])
                d = lax.shift_right_logical(key, sh) & 255
                bn = lanes * 256 + d
                cnt = plsc.load_gather(hist, [bn])
                plsc.store_scatter(hist, [bn], cnt + 1)
            return jnp.int32(0)

        lax.fori_loop(jnp.int32(0), jnp.int32(_PL // 8), h_body, jnp.int32(0))

        def t_body(g, _):
            g = jnp.int32(g)
            acc = jnp.zeros(16, jnp.int32)
            for l in range(16):
                acc = acc + hist[pl.ds(l * 256 + g * 16, 16)]
            wbuf[pl.ds(g * 16, 16)] = acc
            return jnp.int32(0)

        lax.fori_loop(jnp.int32(0), jnp.int32(16), t_body, jnp.int32(0))
        pltpu.sync_copy(wbuf.at[pl.ds(0, 256)], table.at[pl.ds(w * 256, 256)])
        plsc.subcore_barrier()

        pltpu.sync_copy(table, tloc)

        carry = jnp.int32(0)
        for g in range(16):
            tot = jnp.zeros(16, jnp.int32)
            pri = jnp.zeros(16, jnp.int32)
            for t in range(16):
                v = tloc[pl.ds(t * 256 + g * 16, 16)]
                tot = tot + v
                pri = pri + jnp.where(jnp.int32(t) < w, v, jnp.int32(0))
            ex = plsc.cumsum(tot) - tot + carry
            carry = carry + jnp.sum(tot, dtype=jnp.int32)
            rowbase = ex + pri
            lacc = jnp.zeros(16, jnp.int32)
            for l in range(16):
                run[pl.ds(l * 256 + g * 16, 16)] = rowbase + lacc
                lacc = lacc + hist[pl.ds(l * 256 + g * 16, 16)]

        last = p == 3

        def p_body(i, vacc):
            i = jnp.int32(i)
            for q in range(8):
                j = lanes * _PL + (i * 8 + q)
                key = plsc.load_gather(ck, [j])
                d = lax.shift_right_logical(key, sh) & 255
                bn = lanes * 256 + d
                pos = plsc.load_gather(run, [bn])
                plsc.store_scatter(run, [bn], pos + 1)
                plsc.store_scatter(plist, [j], pos)
                if last:
                    vacc = vacc + jnp.where(pos == jnp.int32(_C0), key,
                                            jnp.int32(0))
            return vacc

        vacc = lax.fori_loop(jnp.int32(0), jnp.int32(_PL // 8), p_body,
                             jnp.zeros(16, jnp.int32))

        if not last:
            pltpu.sync_copy(ck, kbuf.at[plist])
        pltpu.sync_copy(ci, ibuf.at[plist])
        plsc.subcore_barrier()
        if not last:
            pltpu.sync_copy(kbuf.at[pl.ds(base, _CH)], ck)
        pltpu.sync_copy(ibuf.at[pl.ds(base, _CH)], ci)

    pltpu.sync_copy(ci, sidx_hbm.at[c].at[pl.ds(base, _CH)])

    vpart = jnp.sum(vacc, dtype=jnp.int32)
    small[...] = jnp.where(lanes == 0, vpart, jnp.int32(0))
    pltpu.sync_copy(small.at[pl.ds(0, 8)], table.at[pl.ds(w * 8, 8)])
    plsc.subcore_barrier()
    pltpu.sync_copy(table.at[pl.ds(0, 128)], tloc.at[pl.ds(0, 128)])
    vstar = jnp.sum(plsc.load_gather(tloc, [lanes * 8]), dtype=jnp.int32)
    vs = vstar ^ jnp.int32(_MIN32)

    def c_body(i, acc):
        i = jnp.int32(i)
        lt, le = acc
        for q in range(4):
            kv = ck[pl.ds((i * 4 + q) * 16, 16)] ^ jnp.int32(_MIN32)
            lt = lt + jnp.where(kv < vs, jnp.int32(1), jnp.int32(0))
            le = le + jnp.where(kv <= vs, jnp.int32(1), jnp.int32(0))
        return (lt, le)

    lt, le = lax.fori_loop(jnp.int32(0), jnp.int32(_CH // 64), c_body,
                           (jnp.zeros(16, jnp.int32), jnp.zeros(16, jnp.int32)))
    sp = jnp.sum(lt, dtype=jnp.int32)
    ep = jnp.sum(le, dtype=jnp.int32)
    small[...] = jnp.where(lanes == 0, sp, jnp.where(lanes == 1, ep, jnp.int32(0)))
    pltpu.sync_copy(small.at[pl.ds(0, 8)], table.at[pl.ds(128 + w * 8, 8)])
    plsc.subcore_barrier()
    pltpu.sync_copy(table.at[pl.ds(128, 128)], tloc.at[pl.ds(0, 128)])
    s_all = jnp.sum(plsc.load_gather(tloc, [lanes * 8]), dtype=jnp.int32)
    e_all = jnp.sum(plsc.load_gather(tloc, [lanes * 8 + 1]), dtype=jnp.int32)
    blk = e_all - jnp.int32(_C0)

    def hu_body(i, _):
        i = jnp.int32(i)
        for q in range(4):
            i4 = i * 4 + q
            gpos = base + i4 * 16 + lanes
            in_hi = gpos >= e_all
            in_tie = (gpos >= s_all) & (gpos < s_all + blk)
            hm = jnp.where(in_hi | in_tie, jnp.int32(1), jnp.int32(0))
            u = (jnp.maximum(jnp.int32(0),
                             jnp.minimum(gpos, s_all + blk) - s_all)
                 + jnp.maximum(jnp.int32(0), gpos - e_all))
            wv = u + hm * jnp.int32(1 << _HB)
            wbuf[pl.ds(i4 * 16, 16)] = wv
        return jnp.int32(0)

    lax.fori_loop(jnp.int32(0), jnp.int32(_CH // 64), hu_body, jnp.int32(0))
    pltpu.sync_copy(wbuf, hubuf.at[ci])
    plsc.subcore_barrier()

    @pl.when(c == 0)
    def _():
        pltpu.sync_copy(hubuf.at[pl.ds(base, _CH)], hu_hbm.at[pl.ds(base, _CH)])


def _sums_body(sidx_hbm, hu_hbm, s1p_hbm, sabp_hbm, av, bv, hv, l1, l2, l3,
               small, dtab, table, hu_s, tloc16, sem):
    c = lax.axis_index("c")
    w = lax.axis_index("s")
    lanes = _lanes()
    base = w * _CH
    cc = jnp.float32((_N - 1) / 2.0)
    mu = jnp.float32((_K - 1) / 2.0)

    @pl.when(c == 0)
    def _():
        pltpu.sync_copy(sidx_hbm.at[jnp.int32(0)].at[pl.ds(base, _CH)], av)
        pltpu.sync_copy(sidx_hbm.at[jnp.int32(1)].at[pl.ds(base, _CH)], bv)

        def s1_body(i, acc):
            i = jnp.int32(i)
            for q in range(4):
                i4 = i * 4 + q
                am = av[pl.ds(i4 * 16, 16)].astype(jnp.float32) - cc
                bm = bv[pl.ds(i4 * 16, 16)].astype(jnp.float32) - cc
                acc = acc + am * bm
            return acc

        acc = lax.fori_loop(jnp.int32(0), jnp.int32(_CH // 64), s1_body, jnp.zeros(16, jnp.float32))
        part = jnp.sum(acc)
        fsm = jnp.where(lanes == 0, part, jnp.float32(0.0))
        small[...] = lax.bitcast_convert_type(fsm, jnp.int32)
        pltpu.sync_copy(small.at[pl.ds(0, 8)], s1p_hbm.at[pl.ds(w * 8, 8)])

    @pl.when(c == 1)
    def _():
        pltpu.sync_copy(sidx_hbm.at[jnp.int32(1)].at[pl.ds(base, _CH)], av)

        @pl.when(w == jnp.int32(0))
        def _():
            pltpu.sync_copy(hu_hbm, hu_s)

        plsc.subcore_barrier()
        pltpu.async_copy(hu_s.at[av], hv, sem).wait()

        def a_body(i, acc):
            i = jnp.int32(i)
            for q in range(4):
                j = lanes * _PL + (i * 4 + q)
                hu = plsc.load_gather(hv, [j])
                acc = acc + lax.shift_right_logical(hu, jnp.int32(_HB))
            return acc

        tot = lax.fori_loop(jnp.int32(0), jnp.int32(_PL // 4), a_body, jnp.zeros(16, jnp.int32))
        tsum = jnp.sum(tot, dtype=jnp.int32)
        small[...] = jnp.where(lanes == 0, tsum, jnp.int32(0))
        pltpu.sync_copy(small.at[pl.ds(0, 8)], table.at[pl.ds(w * 8, 8)])
        plsc.subcore_barrier()
        pltpu.sync_copy(table.at[pl.ds(0, 128)], tloc16)
        tts = plsc.load_gather(tloc16, [lanes * 8])
        tile_off = jnp.sum(jnp.where(lanes < w, tts, jnp.int32(0)), dtype=jnp.int32)
        start = tile_off + plsc.cumsum(tot) - tot

        def b_body(i, run):
            i = jnp.int32(i)
            for q in range(4):
                j = lanes * _PL + (i * 4 + q)
                hu = plsc.load_gather(hv, [j])
                ht = lax.shift_right_logical(hu, jnp.int32(_HB))
                u = hu & jnp.int32((1 << _HB) - 1)
                du = jnp.int32(_K - 1) - u
                spread = jnp.int32(_DUMP) + (j & 2047)
                hd = ht == 1
                plsc.store_scatter(l1, [j], jnp.where(hd, run, spread))
                plsc.store_scatter(l2, [j], du)
                plsc.store_scatter(l3, [j], jnp.where(hd, u, spread))
                run = run + ht
            return run

        lax.fori_loop(jnp.int32(0), jnp.int32(_PL // 4), b_body, start)
        pltpu.sync_copy(l2, dtab.at[l1])
        plsc.subcore_barrier()
        pltpu.sync_copy(dtab.at[l3], bv)

        def c_body(i, acc):
            i = jnp.int32(i)
            for q in range(4):
                i4 = i * 4 + q
                hu = hv[pl.ds(i4 * 16, 16)]
                g = bv[pl.ds(i4 * 16, 16)]
                ht = lax.shift_right_logical(hu, jnp.int32(_HB))
                u = hu & jnp.int32((1 << _HB) - 1)
                du = (jnp.int32(_K - 1) - u).astype(jnp.float32) - mu
                gm = g.astype(jnp.float32) - mu
                acc = acc + jnp.where(ht == 1, du * gm, jnp.float32(0.0))
            return acc

        acc = lax.fori_loop(jnp.int32(0), jnp.int32(_CH // 64), c_body, jnp.zeros(16, jnp.float32))
        part = jnp.sum(acc)
        fsm = jnp.where(lanes == 0, part, jnp.float32(0.0))
        small[...] = lax.bitcast_convert_type(fsm, jnp.int32)
        pltpu.sync_copy(small.at[pl.ds(0, 8)], sabp_hbm.at[pl.ds(w * 8, 8)])


def _combine_kernel(s1_ref, sab_ref, o_ref):
    s1c = jnp.sum(s1_ref[...])
    sabc = jnp.sum(sab_ref[...])
    n = jnp.float32(_N)
    k = jnp.float32(_K)
    s2c = n * (n * n - 1.0) / 12.0
    avg_ic = 1.0 - 6.0 * (2.0 * s2c - 2.0 * s1c) / (n * (n * n - 1.0))
    var = k * (k * k - 1.0) / 12.0
    head_ic = sabc / var
    r0 = avg_ic
    r2 = 2.0 * head_ic
    rmin = jnp.minimum(jnp.minimum(r0, r2), 0.0)
    rmean = (r0 + r2) / 6.0
    o_ref[...] = jnp.full((8, 128), -rmin * 0.5 - rmean * 0.5, jnp.float32)


@jax.jit
def kernel(model_op, tgt_op):
    m = jnp.squeeze(model_op).astype(jnp.float32)
    t = jnp.squeeze(tgt_op).astype(jnp.float32)
    x2 = jnp.stack([m, t])

    mesh = plsc.VectorSubcoreMesh(core_axis_name="c", subcore_axis_name="s")
    rank_k = functools.partial(
        pl.kernel,
        out_type=[
            jax.ShapeDtypeStruct((2, _N), jnp.int32),
            jax.ShapeDtypeStruct((_N,), jnp.int32),
        ],
        mesh=mesh,
        compiler_params=pltpu.CompilerParams(needs_layout_passes=False),
        scratch_types=[
            pltpu.VMEM((_CH,), jnp.float32),
            pltpu.VMEM((_CH,), jnp.int32),
            pltpu.VMEM((_CH,), jnp.int32),
            pltpu.VMEM((4096,), jnp.int32),
            pltpu.VMEM((4096,), jnp.int32),
            pltpu.VMEM((_CH,), jnp.int32),
            pltpu.VMEM((_CH,), jnp.int32),
            pltpu.VMEM((4096,), jnp.int32),
            pltpu.VMEM((16,), jnp.int32),
            pltpu.VMEM_SHARED((_N,), jnp.int32),
            pltpu.VMEM_SHARED((_N,), jnp.int32),
            pltpu.VMEM_SHARED((4096,), jnp.int32),
            pltpu.VMEM_SHARED((_N,), jnp.int32),
            pltpu.SemaphoreType.DMA,
        ],
    )(_rank_body)
    sidx, hu = rank_k(x2)

    sums_k = functools.partial(
        pl.kernel,
        out_type=[
            jax.ShapeDtypeStruct((128,), jnp.int32),
            jax.ShapeDtypeStruct((128,), jnp.int32),
        ],
        mesh=mesh,
        compiler_params=pltpu.CompilerParams(needs_layout_passes=False),
        scratch_types=[
            pltpu.VMEM((_CH,), jnp.int32),
            pltpu.VMEM((_CH,), jnp.int32),
            pltpu.VMEM((_CH,), jnp.int32),
            pltpu.VMEM((_CH,), jnp.int32),
            pltpu.VMEM((_CH,), jnp.int32),
            pltpu.VMEM((_CH,), jnp.int32),
            pltpu.VMEM((16,), jnp.int32),
            pltpu.VMEM_SHARED((_DSZ,), jnp.int32),
            pltpu.VMEM_SHARED((4096,), jnp.int32),
            pltpu.VMEM_SHARED((_N,), jnp.int32),
            pltpu.VMEM((128,), jnp.int32),
            pltpu.SemaphoreType.DMA,
        ],
    )(_sums_body)
    s1p_i, sabp_i = sums_k(sidx, hu)
    s1p = lax.bitcast_convert_type(s1p_i, jnp.float32).reshape(1, 128)
    sabp = lax.bitcast_convert_type(sabp_i, jnp.float32).reshape(1, 128)

    out = pl.pallas_call(
        _combine_kernel,
        out_shape=jax.ShapeDtypeStruct((8, 128), jnp.float32),
    )(s1p, sabp)
    return out[0, 0]


# speedup vs baseline: 1.1472x; 1.0105x over previous
import functools

import jax

jax.config.update("jax_enable_x64", True)

import jax.numpy as jnp
from jax import lax
from jax.experimental import pallas as pl
from jax.experimental.pallas import tpu as pltpu
from jax.experimental.pallas import tpu_sc as plsc

_N = 65536
_K = int(_N * 0.2)
_C0 = _N - _K
_NT = 16
_CH = _N // _NT
_PL = _CH // 16
_HB = 20
_DUMP = 13112
_DSZ = _DUMP + 2048 + 8
_MIN32 = -2147483648


def _lanes():
    return lax.iota(jnp.int32, 16)


def _sortable(b):
    return jnp.where(b < 0, ~b, b ^ jnp.int32(_MIN32))


def _rank_body(x_hbm, sidx_hbm, hu_hbm, xf, ck, ci, hist, run, plist, wbuf,
               tloc, small, kbuf, ibuf, table, hubuf, sem):
    c = lax.axis_index("c")
    w = lax.axis_index("s")
    lanes = _lanes()
    base = w * _CH

    pltpu.sync_copy(x_hbm.at[c].at[pl.ds(base, _CH)], xf)

    def init_body(i, _):
        i = jnp.int32(i)
        for q in range(4):
            i4 = i * 4 + q
            xv = xf[pl.ds(i4 * 16, 16)]
            b = lax.bitcast_convert_type(xv, jnp.int32)
            ck[pl.ds(i4 * 16, 16)] = _sortable(b)
            ci[pl.ds(i4 * 16, 16)] = base + i4 * 16 + lanes
        return jnp.int32(0)

    lax.fori_loop(jnp.int32(0), jnp.int32(_CH // 64), init_body, jnp.int32(0))

    for p in range(4):
        sh = jnp.int32(8 * p)

        def zero_body(i, _):
            i = jnp.int32(i)
            for q in range(8):
                hist[pl.ds((i * 8 + q) * 16, 16)] = jnp.zeros(16, jnp.int32)
            return jnp.int32(0)

        lax.fori_loop(jnp.int32(0), jnp.int32(_CH // 128), zero_body,
                      jnp.int32(0))

        def h_body(i, _):
            i = jnp.int32(i)
            for q in range(8):
                j = lanes * _PL + (i * 8 + q)
                key = plsc.load_gather(ck, [j])
                d = lax.shift_right_logical(key, sh) & 255
                bn = lanes * 256 + d
                cnt = plsc.load_gather(hist, [bn])
                plsc.store_scatter(hist, [bn], cnt + 1)
            return jnp.int32(0)

        lax.fori_loop(jnp.int32(0), jnp.int32(_PL // 8), h_body, jnp.int32(0))

        def t_body(g, _):
            g = jnp.int32(g)
            acc = jnp.zeros(16, jnp.int32)
            for l in range(16):
                acc = acc + hist[pl.ds(l * 256 + g * 16, 16)]
            wbuf[pl.ds(g * 16, 16)] = acc
            return jnp.int32(0)

        lax.fori_loop(jnp.int32(0), jnp.int32(16), t_body, jnp.int32(0))
        pltpu.sync_copy(wbuf.at[pl.ds(0, 256)], table.at[pl.ds(w * 256, 256)])
        plsc.subcore_barrier()

        pltpu.sync_copy(table, tloc)

        carry = jnp.int32(0)
        for g in range(16):
            tot = jnp.zeros(16, jnp.int32)
            pri = jnp.zeros(16, jnp.int32)
            for t in range(16):
                v = tloc[pl.ds(t * 256 + g * 16, 16)]
                tot = tot + v
                pri = pri + jnp.where(jnp.int32(t) < w, v, jnp.int32(0))
            ex = plsc.cumsum(tot) - tot + carry
            carry = carry + jnp.sum(tot, dtype=jnp.int32)
            rowbase = ex + pri
            lacc = jnp.zeros(16, jnp.int32)
            for l in range(16):
                run[pl.ds(l * 256 + g * 16, 16)] = rowbase + lacc
                lacc = lacc + hist[pl.ds(l * 256 + g * 16, 16)]

        last = p == 3

        def p_body(i, vacc):
            i = jnp.int32(i)
            for q in range(8):
                j = lanes * _PL + (i * 8 + q)
                key = plsc.load_gather(ck, [j])
                d = lax.shift_right_logical(key, sh) & 255
                bn = lanes * 256 + d
                pos = plsc.load_gather(run, [bn])
                plsc.store_scatter(run, [bn], pos + 1)
                plsc.store_scatter(plist, [j], pos)
                if last:
                    vacc = vacc + jnp.where(pos == jnp.int32(_C0), key,
                                            jnp.int32(0))
            return vacc

        vacc = lax.fori_loop(jnp.int32(0), jnp.int32(_PL // 8), p_body,
                             jnp.zeros(16, jnp.int32))

        if not last:
            pltpu.sync_copy(ck, kbuf.at[plist])
        pltpu.sync_copy(ci, ibuf.at[plist])
        plsc.subcore_barrier()
        if not last:
            pltpu.sync_copy(kbuf.at[pl.ds(base, _CH)], ck)
        pltpu.sync_copy(ibuf.at[pl.ds(base, _CH)], ci)

    pltpu.sync_copy(ci, sidx_hbm.at[c].at[pl.ds(base, _CH)])

    vpart = jnp.sum(vacc, dtype=jnp.int32)
    small[...] = jnp.where(lanes == 0, vpart, jnp.int32(0))
    pltpu.sync_copy(small.at[pl.ds(0, 8)], table.at[pl.ds(w * 8, 8)])
    plsc.subcore_barrier()
    pltpu.sync_copy(table.at[pl.ds(0, 128)], tloc.at[pl.ds(0, 128)])
    vstar = jnp.sum(plsc.load_gather(tloc, [lanes * 8]), dtype=jnp.int32)
    vs = vstar ^ jnp.int32(_MIN32)

    def c_body(i, acc):
        i = jnp.int32(i)
        lt, le = acc
        for q in range(4):
            kv = ck[pl.ds((i * 4 + q) * 16, 16)] ^ jnp.int32(_MIN32)
            lt = lt + jnp.where(kv < vs, jnp.int32(1), jnp.int32(0))
            le = le + jnp.where(kv <= vs, jnp.int32(1), jnp.int32(0))
        return (lt, le)

    lt, le = lax.fori_loop(jnp.int32(0), jnp.int32(_CH // 64), c_body,
                           (jnp.zeros(16, jnp.int32), jnp.zeros(16, jnp.int32)))
    sp = jnp.sum(lt, dtype=jnp.int32)
    ep = jnp.sum(le, dtype=jnp.int32)
    small[...] = jnp.where(lanes == 0, sp, jnp.where(lanes == 1, ep, jnp.int32(0)))
    pltpu.sync_copy(small.at[pl.ds(0, 8)], table.at[pl.ds(128 + w * 8, 8)])
    plsc.subcore_barrier()
    pltpu.sync_copy(table.at[pl.ds(128, 128)], tloc.at[pl.ds(0, 128)])
    s_all = jnp.sum(plsc.load_gather(tloc, [lanes * 8]), dtype=jnp.int32)
    e_all = jnp.sum(plsc.load_gather(tloc, [lanes * 8 + 1]), dtype=jnp.int32)
    blk = e_all - jnp.int32(_C0)

    def hu_body(i, _):
        i = jnp.int32(i)
        for q in range(4):
            i4 = i * 4 + q
            gpos = base + i4 * 16 + lanes
            in_hi = gpos >= e_all
            in_tie = (gpos >= s_all) & (gpos < s_all + blk)
            hm = jnp.where(in_hi | in_tie, jnp.int32(1), jnp.int32(0))
            u = (jnp.maximum(jnp.int32(0),
                             jnp.minimum(gpos, s_all + blk) - s_all)
                 + jnp.maximum(jnp.int32(0), gpos - e_all))
            wv = u + hm * jnp.int32(1 << _HB)
            wbuf[pl.ds(i4 * 16, 16)] = wv
        return jnp.int32(0)

    lax.fori_loop(jnp.int32(0), jnp.int32(_CH // 64), hu_body, jnp.int32(0))
    pltpu.sync_copy(wbuf, hubuf.at[ci])
    plsc.subcore_barrier()

    @pl.when(c == 0)
    def _():
        pltpu.sync_copy(hubuf.at[pl.ds(base, _CH)], hu_hbm.at[pl.ds(base, _CH)])


def _sums_body(sidx_hbm, hu_hbm, s1p_hbm, sabp_hbm, av, bv, hv, l1, l2, l3,
               small, fsmall, dtab, table, hu_s, tloc16, sem):
    c = lax.axis_index("c")
    w = lax.axis_index("s")
    lanes = _lanes()
    base = w * _CH
    cc = jnp.float32((_N - 1) / 2.0)
    mu = jnp.float32((_K - 1) / 2.0)

    @pl.when(c == 0)
    def _():
        pltpu.sync_copy(sidx_hbm.at[jnp.int32(0)].at[pl.ds(base, _CH)], av)
        pltpu.sync_copy(sidx_hbm.at[jnp.int32(1)].at[pl.ds(base, _CH)], bv)

        def s1_body(i, acc):
            i = jnp.int32(i)
            for q in range(4):
                i4 = i * 4 + q
                am = av[pl.ds(i4 * 16, 16)].astype(jnp.float32) - cc
                bm = bv[pl.ds(i4 * 16, 16)].astype(jnp.float32) - cc
                acc = acc + am * bm
            return acc

        acc = lax.fori_loop(jnp.int32(0), jnp.int32(_CH // 64), s1_body, jnp.zeros(16, jnp.float32))
        part = jnp.sum(acc)
        fsmall[...] = jnp.where(lanes == 0, part, jnp.float32(0.0))
        pltpu.sync_copy(fsmall.at[pl.ds(0, 8)], s1p_hbm.at[pl.ds(w * 8, 8)])

    @pl.when(c == 1)
    def _():
        pltpu.sync_copy(sidx_hbm.at[jnp.int32(1)].at[pl.ds(base, _CH)], av)

        @pl.when(w == jnp.int32(0))
        def _():
            pltpu.sync_copy(hu_hbm, hu_s)

        plsc.subcore_barrier()
        pltpu.async_copy(hu_s.at[av], hv, sem).wait()

        def a_body(i, acc):
            i = jnp.int32(i)
            for q in range(4):
                j = lanes * _PL + (i * 4 + q)
                hu = plsc.load_gather(hv, [j])
                acc = acc + lax.shift_right_logical(hu, jnp.int32(_HB))
            return acc

        tot = lax.fori_loop(jnp.int32(0), jnp.int32(_PL // 4), a_body, jnp.zeros(16, jnp.int32))
        tsum = jnp.sum(tot, dtype=jnp.int32)
        small[...] = jnp.where(lanes == 0, tsum, jnp.int32(0))
        pltpu.sync_copy(small.at[pl.ds(0, 8)], table.at[pl.ds(w * 8, 8)])
        plsc.subcore_barrier()
        pltpu.sync_copy(table.at[pl.ds(0, 128)], tloc16)
        tts = plsc.load_gather(tloc16, [lanes * 8])
        tile_off = jnp.sum(jnp.where(lanes < w, tts, jnp.int32(0)), dtype=jnp.int32)
        start = tile_off + plsc.cumsum(tot) - tot

        def b_body(i, run):
            i = jnp.int32(i)
            for q in range(4):
                j = lanes * _PL + (i * 4 + q)
                hu = plsc.load_gather(hv, [j])
                ht = lax.shift_right_logical(hu, jnp.int32(_HB))
                u = hu & jnp.int32((1 << _HB) - 1)
                du = jnp.int32(_K - 1) - u
                spread = jnp.int32(_DUMP) + (j & 2047)
                hd = ht == 1
                plsc.store_scatter(l1, [j], jnp.where(hd, run, spread))
                plsc.store_scatter(l2, [j], du)
                plsc.store_scatter(l3, [j], jnp.where(hd, u, spread))
                run = run + ht
            return run

        lax.fori_loop(jnp.int32(0), jnp.int32(_PL // 4), b_body, start)
        pltpu.sync_copy(l2, dtab.at[l1])
        plsc.subcore_barrier()
        pltpu.sync_copy(dtab.at[l3], bv)

        def c_body(i, acc):
            i = jnp.int32(i)
            for q in range(4):
                i4 = i * 4 + q
                hu = hv[pl.ds(i4 * 16, 16)]
                g = bv[pl.ds(i4 * 16, 16)]
                ht = lax.shift_right_logical(hu, jnp.int32(_HB))
                u = hu & jnp.int32((1 << _HB) - 1)
                du = (jnp.int32(_K - 1) - u).astype(jnp.float32) - mu
                gm = g.astype(jnp.float32) - mu
                acc = acc + jnp.where(ht == 1, du * gm, jnp.float32(0.0))
            return acc

        acc = lax.fori_loop(jnp.int32(0), jnp.int32(_CH // 64), c_body, jnp.zeros(16, jnp.float32))
        part = jnp.sum(acc)
        fsmall[...] = jnp.where(lanes == 0, part, jnp.float32(0.0))
        pltpu.sync_copy(fsmall.at[pl.ds(0, 8)], sabp_hbm.at[pl.ds(w * 8, 8)])


def _combine_kernel(s1_ref, sab_ref, o_ref):
    s1c = jnp.sum(s1_ref[...])
    sabc = jnp.sum(sab_ref[...])
    n = jnp.float32(_N)
    k = jnp.float32(_K)
    s2c = n * (n * n - 1.0) / 12.0
    avg_ic = 1.0 - 6.0 * (2.0 * s2c - 2.0 * s1c) / (n * (n * n - 1.0))
    var = k * (k * k - 1.0) / 12.0
    head_ic = sabc / var
    r0 = avg_ic
    r2 = 2.0 * head_ic
    rmin = jnp.minimum(jnp.minimum(r0, r2), 0.0)
    rmean = (r0 + r2) / 6.0
    o_ref[...] = jnp.full((8, 128), -rmin * 0.5 - rmean * 0.5, jnp.float32)


@jax.jit
def kernel(model_op, tgt_op):
    m = jnp.squeeze(model_op).astype(jnp.float32)
    t = jnp.squeeze(tgt_op).astype(jnp.float32)
    x2 = jnp.stack([m, t])

    mesh = plsc.VectorSubcoreMesh(core_axis_name="c", subcore_axis_name="s")
    rank_k = functools.partial(
        pl.kernel,
        out_type=[
            jax.ShapeDtypeStruct((2, _N), jnp.int32),
            jax.ShapeDtypeStruct((_N,), jnp.int32),
        ],
        mesh=mesh,
        compiler_params=pltpu.CompilerParams(needs_layout_passes=False),
        scratch_types=[
            pltpu.VMEM((_CH,), jnp.float32),
            pltpu.VMEM((_CH,), jnp.int32),
            pltpu.VMEM((_CH,), jnp.int32),
            pltpu.VMEM((4096,), jnp.int32),
            pltpu.VMEM((4096,), jnp.int32),
            pltpu.VMEM((_CH,), jnp.int32),
            pltpu.VMEM((_CH,), jnp.int32),
            pltpu.VMEM((4096,), jnp.int32),
            pltpu.VMEM((16,), jnp.int32),
            pltpu.VMEM_SHARED((_N,), jnp.int32),
            pltpu.VMEM_SHARED((_N,), jnp.int32),
            pltpu.VMEM_SHARED((4096,), jnp.int32),
            pltpu.VMEM_SHARED((_N,), jnp.int32),
            pltpu.SemaphoreType.DMA,
        ],
    )(_rank_body)
    sidx, hu = rank_k(x2)

    sums_k = functools.partial(
        pl.kernel,
        out_type=[
            jax.ShapeDtypeStruct((128,), jnp.float32),
            jax.ShapeDtypeStruct((128,), jnp.float32),
        ],
        mesh=mesh,
        compiler_params=pltpu.CompilerParams(needs_layout_passes=False),
        scratch_types=[
            pltpu.VMEM((_CH,), jnp.int32),
            pltpu.VMEM((_CH,), jnp.int32),
            pltpu.VMEM((_CH,), jnp.int32),
            pltpu.VMEM((_CH,), jnp.int32),
            pltpu.VMEM((_CH,), jnp.int32),
            pltpu.VMEM((_CH,), jnp.int32),
            pltpu.VMEM((16,), jnp.int32),
            pltpu.VMEM((16,), jnp.float32),
            pltpu.VMEM_SHARED((_DSZ,), jnp.int32),
            pltpu.VMEM_SHARED((4096,), jnp.int32),
            pltpu.VMEM_SHARED((_N,), jnp.int32),
            pltpu.VMEM((128,), jnp.int32),
            pltpu.SemaphoreType.DMA,
        ],
    )(_sums_body)
    s1p, sabp = sums_k(sidx, hu)
    s1p = s1p.reshape(1, 128)
    sabp = sabp.reshape(1, 128)

    out = pl.pallas_call(
        _combine_kernel,
        out_shape=jax.ShapeDtypeStruct((8, 128), jnp.float32),
    )(s1p, sabp)
    return out[0, 0]


# speedup vs baseline: 1.1482x; 1.0008x over previous
import functools

import jax

jax.config.update("jax_enable_x64", True)

import jax.numpy as jnp
from jax import lax
from jax.experimental import pallas as pl
from jax.experimental.pallas import tpu as pltpu
from jax.experimental.pallas import tpu_sc as plsc

_N = 65536
_K = int(_N * 0.2)
_C0 = _N - _K
_NT = 16
_CH = _N // _NT
_PL = _CH // 16
_HB = 20
_DUMP = 13112
_DSZ = _DUMP + 2048 + 8
_MIN32 = -2147483648


def _lanes():
    return lax.iota(jnp.int32, 16)


def _sortable(b):
    return jnp.where(b < 0, ~b, b ^ jnp.int32(_MIN32))


def _rank_body(x_hbm, sidx_hbm, hu_hbm, xf, ck, ci, hist, run, plist, wbuf,
               tloc, small, kbuf, ibuf, table, hubuf, sem):
    c = lax.axis_index("c")
    w = lax.axis_index("s")
    lanes = _lanes()
    base = w * _CH

    pltpu.sync_copy(x_hbm.at[c].at[pl.ds(base, _CH)], xf)

    def init_body(i, _):
        i = jnp.int32(i)
        for q in range(4):
            i4 = i * 4 + q
            xv = xf[pl.ds(i4 * 16, 16)]
            b = lax.bitcast_convert_type(xv, jnp.int32)
            ck[pl.ds(i4 * 16, 16)] = _sortable(b)
            ci[pl.ds(i4 * 16, 16)] = base + i4 * 16 + lanes
        return jnp.int32(0)

    lax.fori_loop(jnp.int32(0), jnp.int32(_CH // 64), init_body, jnp.int32(0))

    for p in range(4):
        sh = jnp.int32(8 * p)

        def zero_body(i, _):
            i = jnp.int32(i)
            for q in range(8):
                hist[pl.ds((i * 8 + q) * 16, 16)] = jnp.zeros(16, jnp.int32)
            return jnp.int32(0)

        lax.fori_loop(jnp.int32(0), jnp.int32(_CH // 128), zero_body,
                      jnp.int32(0))

        def h_body(i, _):
            i = jnp.int32(i)
            for q in range(8):
                j = lanes * _PL + (i * 8 + q)
                key = plsc.load_gather(ck, [j])
                d = lax.shift_right_logical(key, sh) & 255
                bn = lanes * 256 + d
                cnt = plsc.load_gather(hist, [bn])
                plsc.store_scatter(hist, [bn], cnt + 1)
            return jnp.int32(0)

        lax.fori_loop(jnp.int32(0), jnp.int32(_PL // 8), h_body, jnp.int32(0))

        def t_body(g, _):
            g = jnp.int32(g)
            acc = jnp.zeros(16, jnp.int32)
            for l in range(16):
                acc = acc + hist[pl.ds(l * 256 + g * 16, 16)]
            wbuf[pl.ds(g * 16, 16)] = acc
            return jnp.int32(0)

        lax.fori_loop(jnp.int32(0), jnp.int32(16), t_body, jnp.int32(0))
        pltpu.sync_copy(wbuf.at[pl.ds(0, 256)], table.at[pl.ds(w * 256, 256)])
        plsc.subcore_barrier()

        pltpu.sync_copy(table, tloc)

        carry = jnp.int32(0)
        for g in range(16):
            tot = jnp.zeros(16, jnp.int32)
            pri = jnp.zeros(16, jnp.int32)
            for t in range(16):
                v = tloc[pl.ds(t * 256 + g * 16, 16)]
                tot = tot + v
                pri = pri + jnp.where(jnp.int32(t) < w, v, jnp.int32(0))
            ex = plsc.cumsum(tot) - tot + carry
            carry = carry + jnp.sum(tot, dtype=jnp.int32)
            rowbase = ex + pri
            lacc = jnp.zeros(16, jnp.int32)
            for l in range(16):
                run[pl.ds(l * 256 + g * 16, 16)] = rowbase + lacc
                lacc = lacc + hist[pl.ds(l * 256 + g * 16, 16)]

        last = p == 3

        def p_body(i, vacc):
            i = jnp.int32(i)
            for q in range(8):
                j = lanes * _PL + (i * 8 + q)
                key = plsc.load_gather(ck, [j])
                d = lax.shift_right_logical(key, sh) & 255
                bn = lanes * 256 + d
                pos = plsc.load_gather(run, [bn])
                plsc.store_scatter(run, [bn], pos + 1)
                plsc.store_scatter(plist, [j], pos)
                if last:
                    vacc = vacc + jnp.where(pos == jnp.int32(_C0), key,
                                            jnp.int32(0))
            return vacc

        vacc = lax.fori_loop(jnp.int32(0), jnp.int32(_PL // 8), p_body,
                             jnp.zeros(16, jnp.int32))

        if not last:
            pltpu.sync_copy(ck, kbuf.at[plist])
        pltpu.sync_copy(ci, ibuf.at[plist])
        plsc.subcore_barrier()
        if not last:
            pltpu.sync_copy(kbuf.at[pl.ds(base, _CH)], ck)
        pltpu.sync_copy(ibuf.at[pl.ds(base, _CH)], ci)

    pltpu.sync_copy(ci, sidx_hbm.at[c].at[pl.ds(base, _CH)])

    vpart = jnp.sum(vacc, dtype=jnp.int32)
    small[...] = jnp.where(lanes == 0, vpart, jnp.int32(0))
    pltpu.sync_copy(small.at[pl.ds(0, 8)], table.at[pl.ds(w * 8, 8)])
    plsc.subcore_barrier()
    pltpu.sync_copy(table.at[pl.ds(0, 128)], tloc.at[pl.ds(0, 128)])
    vstar = jnp.sum(plsc.load_gather(tloc, [lanes * 8]), dtype=jnp.int32)
    vs = vstar ^ jnp.int32(_MIN32)

    def c_body(i, acc):
        i = jnp.int32(i)
        lt, le = acc
        for q in range(4):
            kv = ck[pl.ds((i * 4 + q) * 16, 16)] ^ jnp.int32(_MIN32)
            lt = lt + jnp.where(kv < vs, jnp.int32(1), jnp.int32(0))
            le = le + jnp.where(kv <= vs, jnp.int32(1), jnp.int32(0))
        return (lt, le)

    lt, le = lax.fori_loop(jnp.int32(0), jnp.int32(_CH // 64), c_body,
                           (jnp.zeros(16, jnp.int32), jnp.zeros(16, jnp.int32)))
    sp = jnp.sum(lt, dtype=jnp.int32)
    ep = jnp.sum(le, dtype=jnp.int32)
    small[...] = jnp.where(lanes == 0, sp, jnp.where(lanes == 1, ep, jnp.int32(0)))
    pltpu.sync_copy(small.at[pl.ds(0, 8)], table.at[pl.ds(128 + w * 8, 8)])
    plsc.subcore_barrier()
    pltpu.sync_copy(table.at[pl.ds(128, 128)], tloc.at[pl.ds(0, 128)])
    s_all = jnp.sum(plsc.load_gather(tloc, [lanes * 8]), dtype=jnp.int32)
    e_all = jnp.sum(plsc.load_gather(tloc, [lanes * 8 + 1]), dtype=jnp.int32)
    blk = e_all - jnp.int32(_C0)

    def hu_body(i, _):
        i = jnp.int32(i)
        for q in range(4):
            i4 = i * 4 + q
            gpos = base + i4 * 16 + lanes
            in_hi = gpos >= e_all
            in_tie = (gpos >= s_all) & (gpos < s_all + blk)
            hm = jnp.where(in_hi | in_tie, jnp.int32(1), jnp.int32(0))
            u = (jnp.maximum(jnp.int32(0),
                             jnp.minimum(gpos, s_all + blk) - s_all)
                 + jnp.maximum(jnp.int32(0), gpos - e_all))
            wv = u + hm * jnp.int32(1 << _HB)
            wbuf[pl.ds(i4 * 16, 16)] = wv
        return jnp.int32(0)

    lax.fori_loop(jnp.int32(0), jnp.int32(_CH // 64), hu_body, jnp.int32(0))
    pltpu.sync_copy(wbuf, hubuf.at[ci])
    plsc.subcore_barrier()

    @pl.when(c == 0)
    def _():
        pltpu.sync_copy(hubuf.at[pl.ds(base, _CH)], hu_hbm.at[pl.ds(base, _CH)])


def _sums_body(sidx_hbm, hu_hbm, s1p_hbm, sabp_hbm, av, bv, hv, l1, l2, l3,
               small, fsmall, dtab, table, hu_s, tloc16, sem):
    c = lax.axis_index("c")
    w = lax.axis_index("s")
    lanes = _lanes()
    base = w * _CH
    cc = jnp.float32((_N - 1) / 2.0)
    mu = jnp.float32((_K - 1) / 2.0)

    @pl.when(c == 0)
    def _():
        pltpu.sync_copy(sidx_hbm.at[jnp.int32(0)].at[pl.ds(base, _CH)], av)
        pltpu.sync_copy(sidx_hbm.at[jnp.int32(1)].at[pl.ds(base, _CH)], bv)

        def s1_body(i, acc):
            i = jnp.int32(i)
            for q in range(4):
                i4 = i * 4 + q
                am = av[pl.ds(i4 * 16, 16)].astype(jnp.float32) - cc
                bm = bv[pl.ds(i4 * 16, 16)].astype(jnp.float32) - cc
                acc = acc + am * bm
            return acc

        acc = lax.fori_loop(jnp.int32(0), jnp.int32(_CH // 64), s1_body, jnp.zeros(16, jnp.float32))
        part = jnp.sum(acc)
        fsmall[...] = jnp.where(lanes == 0, part, jnp.float32(0.0))
        pltpu.sync_copy(fsmall.at[pl.ds(0, 8)], s1p_hbm.at[pl.ds(w * 8, 8)])

    @pl.when(c == 1)
    def _():
        pltpu.sync_copy(sidx_hbm.at[jnp.int32(1)].at[pl.ds(base, _CH)], av)

        pltpu.sync_copy(hu_hbm.at[pl.ds(base, _CH)], hu_s.at[pl.ds(base, _CH)])
        plsc.subcore_barrier()
        pltpu.async_copy(hu_s.at[av], hv, sem).wait()

        def a_body(i, acc):
            i = jnp.int32(i)
            for q in range(4):
                j = lanes * _PL + (i * 4 + q)
                hu = plsc.load_gather(hv, [j])
                acc = acc + lax.shift_right_logical(hu, jnp.int32(_HB))
            return acc

        tot = lax.fori_loop(jnp.int32(0), jnp.int32(_PL // 4), a_body, jnp.zeros(16, jnp.int32))
        tsum = jnp.sum(tot, dtype=jnp.int32)
        small[...] = jnp.where(lanes == 0, tsum, jnp.int32(0))
        pltpu.sync_copy(small.at[pl.ds(0, 8)], table.at[pl.ds(w * 8, 8)])
        plsc.subcore_barrier()
        pltpu.sync_copy(table.at[pl.ds(0, 128)], tloc16)
        tts = plsc.load_gather(tloc16, [lanes * 8])
        tile_off = jnp.sum(jnp.where(lanes < w, tts, jnp.int32(0)), dtype=jnp.int32)
        start = tile_off + plsc.cumsum(tot) - tot

        def b_body(i, run):
            i = jnp.int32(i)
            for q in range(4):
                j = lanes * _PL + (i * 4 + q)
                hu = plsc.load_gather(hv, [j])
                ht = lax.shift_right_logical(hu, jnp.int32(_HB))
                u = hu & jnp.int32((1 << _HB) - 1)
                du = jnp.int32(_K - 1) - u
                spread = jnp.int32(_DUMP) + (j & 2047)
                hd = ht == 1
                plsc.store_scatter(l1, [j], jnp.where(hd, run, spread))
                plsc.store_scatter(l2, [j], du)
                plsc.store_scatter(l3, [j], jnp.where(hd, u, spread))
                run = run + ht
            return run

        lax.fori_loop(jnp.int32(0), jnp.int32(_PL // 4), b_body, start)
        pltpu.sync_copy(l2, dtab.at[l1])
        plsc.subcore_barrier()
        pltpu.sync_copy(dtab.at[l3], bv)

        def c_body(i, acc):
            i = jnp.int32(i)
            for q in range(4):
                i4 = i * 4 + q
                hu = hv[pl.ds(i4 * 16, 16)]
                g = bv[pl.ds(i4 * 16, 16)]
                ht = lax.shift_right_logical(hu, jnp.int32(_HB))
                u = hu & jnp.int32((1 << _HB) - 1)
                du = (jnp.int32(_K - 1) - u).astype(jnp.float32) - mu
                gm = g.astype(jnp.float32) - mu
                acc = acc + jnp.where(ht == 1, du * gm, jnp.float32(0.0))
            return acc

        acc = lax.fori_loop(jnp.int32(0), jnp.int32(_CH // 64), c_body, jnp.zeros(16, jnp.float32))
        part = jnp.sum(acc)
        fsmall[...] = jnp.where(lanes == 0, part, jnp.float32(0.0))
        pltpu.sync_copy(fsmall.at[pl.ds(0, 8)], sabp_hbm.at[pl.ds(w * 8, 8)])


def _combine_kernel(s1_ref, sab_ref, o_ref):
    s1c = jnp.sum(s1_ref[...])
    sabc = jnp.sum(sab_ref[...])
    n = jnp.float32(_N)
    k = jnp.float32(_K)
    s2c = n * (n * n - 1.0) / 12.0
    avg_ic = 1.0 - 6.0 * (2.0 * s2c - 2.0 * s1c) / (n * (n * n - 1.0))
    var = k * (k * k - 1.0) / 12.0
    head_ic = sabc / var
    r0 = avg_ic
    r2 = 2.0 * head_ic
    rmin = jnp.minimum(jnp.minimum(r0, r2), 0.0)
    rmean = (r0 + r2) / 6.0
    o_ref[...] = jnp.full((8, 128), -rmin * 0.5 - rmean * 0.5, jnp.float32)


@jax.jit
def kernel(model_op, tgt_op):
    m = jnp.squeeze(model_op).astype(jnp.float32)
    t = jnp.squeeze(tgt_op).astype(jnp.float32)
    x2 = jnp.stack([m, t])

    mesh = plsc.VectorSubcoreMesh(core_axis_name="c", subcore_axis_name="s")
    rank_k = functools.partial(
        pl.kernel,
        out_type=[
            jax.ShapeDtypeStruct((2, _N), jnp.int32),
            jax.ShapeDtypeStruct((_N,), jnp.int32),
        ],
        mesh=mesh,
        compiler_params=pltpu.CompilerParams(needs_layout_passes=False),
        scratch_types=[
            pltpu.VMEM((_CH,), jnp.float32),
            pltpu.VMEM((_CH,), jnp.int32),
            pltpu.VMEM((_CH,), jnp.int32),
            pltpu.VMEM((4096,), jnp.int32),
            pltpu.VMEM((4096,), jnp.int32),
            pltpu.VMEM((_CH,), jnp.int32),
            pltpu.VMEM((_CH,), jnp.int32),
            pltpu.VMEM((4096,), jnp.int32),
            pltpu.VMEM((16,), jnp.int32),
            pltpu.VMEM_SHARED((_N,), jnp.int32),
            pltpu.VMEM_SHARED((_N,), jnp.int32),
            pltpu.VMEM_SHARED((4096,), jnp.int32),
            pltpu.VMEM_SHARED((_N,), jnp.int32),
            pltpu.SemaphoreType.DMA,
        ],
    )(_rank_body)
    sidx, hu = rank_k(x2)

    sums_k = functools.partial(
        pl.kernel,
        out_type=[
            jax.ShapeDtypeStruct((128,), jnp.float32),
            jax.ShapeDtypeStruct((128,), jnp.float32),
        ],
        mesh=mesh,
        compiler_params=pltpu.CompilerParams(needs_layout_passes=False),
        scratch_types=[
            pltpu.VMEM((_CH,), jnp.int32),
            pltpu.VMEM((_CH,), jnp.int32),
            pltpu.VMEM((_CH,), jnp.int32),
            pltpu.VMEM((_CH,), jnp.int32),
            pltpu.VMEM((_CH,), jnp.int32),
            pltpu.VMEM((_CH,), jnp.int32),
            pltpu.VMEM((16,), jnp.int32),
            pltpu.VMEM((16,), jnp.float32),
            pltpu.VMEM_SHARED((_DSZ,), jnp.int32),
            pltpu.VMEM_SHARED((4096,), jnp.int32),
            pltpu.VMEM_SHARED((_N,), jnp.int32),
            pltpu.VMEM((128,), jnp.int32),
            pltpu.SemaphoreType.DMA,
        ],
    )(_sums_body)
    s1p, sabp = sums_k(sidx, hu)
    s1p = s1p.reshape(1, 128)
    sabp = sabp.reshape(1, 128)

    out = pl.pallas_call(
        _combine_kernel,
        out_shape=jax.ShapeDtypeStruct((8, 128), jnp.float32),
    )(s1p, sabp)
    return out[0, 0]
